# baseline (speedup 1.0000x reference)
import jax
import jax.numpy as jnp
from jax.experimental import pallas as pl
from jax.experimental.pallas import tpu as pltpu
from jax.experimental.pallas import tpu_sc as plsc

_N_E = 8192
_D = 256
_N_TOK = 4608
_BM = 512
_BN = 1024
_NI = _N_TOK // _BM
_NJ = _N_E // _BN
_GW = 128
_EPS = 1e-12


def _bfly_rowsum(s):
    for sh in (64, 32, 16, 8, 4, 2, 1):
        s = s + pltpu.roll(s, sh, 1)
    return s


def _argmin_body(z_ref, e_ref, idx_ref, loss_ref, enb_s):
    i = pl.program_id(0)

    @pl.when(i == 0)
    def _prep_codebook():
        lane0 = jax.lax.broadcasted_iota(jnp.int32, (_BN, 128), 1) == 0
        for j in range(_NJ):
            e = e_ref[j * _BN:(j + 1) * _BN, :]
            e_lo, e_hi = e[:, :128], e[:, 128:]
            nsq = _bfly_rowsum(e_lo * e_lo + e_hi * e_hi)
            m = jnp.maximum(jnp.sqrt(nsq), _EPS)
            en_lo, en_hi = e_lo / m, e_hi / m
            esqm1 = _bfly_rowsum(en_lo * en_lo + en_hi * en_hi) - 1.0
            ext = jnp.where(lane0, esqm1, jnp.zeros_like(esqm1))
            enb_s[j] = jnp.concatenate([en_lo, en_hi, ext],
                                       axis=1).astype(jnp.bfloat16)

    zb = z_ref[...]
    nrm = jnp.sqrt(jnp.sum(zb * zb, axis=1, keepdims=True))
    zn = zb / jnp.maximum(nrm, _EPS)
    zsq = jnp.sum(zn * zn, keepdims=True)
    zlane0 = jax.lax.broadcasted_iota(jnp.int32, (_BM, 128), 1) == 0
    zext = jnp.where(zlane0, jnp.full((_BM, 128), 1.0, jnp.float32),
                     jnp.zeros((_BM, 128), jnp.float32))
    znm2 = jnp.concatenate([-2.0 * zn, zext],
                           axis=1).astype(jnp.bfloat16)

    nr = _BM // 128
    rv = [None] * nr
    ri = [None] * nr
    for j in range(_NJ):
        score = jax.lax.dot_general(
            znm2, enb_s[j],
            dimension_numbers=(((1,), (1,)), ((), ())),
            preferred_element_type=jnp.float32)
        for r in range(nr):
            rvc, ric = rv[r], ri[r]
            for g in range(_BN // 128):
                sg = score[r * 128:(r + 1) * 128, g * 128:(g + 1) * 128]
                fold = j * (_BN // 128) + g
                if rvc is None:
                    rvc = sg
                    ric = jnp.zeros((128, 128), jnp.int32)
                else:
                    m = sg < rvc
                    rvc = jnp.where(m, sg, rvc)
                    ric = jnp.where(m, jnp.full((128, 128), fold,
                                                jnp.int32), ric)
            rv[r], ri[r] = rvc, ric

    rva = jnp.concatenate(rv, axis=0)
    ria = jnp.concatenate(ri, axis=0)
    rvt = rva.T
    rit = ria.T
    sub = jax.lax.broadcasted_iota(jnp.int32, (128, _BM), 0)
    gidx = rit * 128 + sub
    mv = jnp.min(rvt, axis=0, keepdims=True)
    cand = jnp.where(rvt == mv, gidx, jnp.full((128, _BM), 2**31 - 1,
                                               jnp.int32))
    idx_ref[...] = jnp.min(cand, axis=0, keepdims=True).reshape(1, 1, _BM)
    part = zsq + (jnp.sum(mv, keepdims=True) + float(_BM))
    prev = jnp.where(i == 0, jnp.zeros((1, 1), jnp.float32),
                     loss_ref[...])
    tot = prev + part
    scale = 1.25 / float(_N_TOK * _D)
    tot = jnp.where(i == _NI - 1, tot * scale, tot)
    loss_ref[...] = tot


def _argmin_call(z_flat, emb):
    return pl.pallas_call(
        _argmin_body,
        grid=(_NI,),
        in_specs=[
            pl.BlockSpec((_BM, _D), lambda i: (i, 0)),
            pl.BlockSpec((_N_E, _D), lambda i: (0, 0)),
        ],
        out_specs=[
            pl.BlockSpec((1, 1, _BM), lambda i: (i, 0, 0)),
            pl.BlockSpec((1, 1), lambda i: (0, 0)),
        ],
        out_shape=[
            jax.ShapeDtypeStruct((_NI, 1, _BM), jnp.int32),
            jax.ShapeDtypeStruct((1, 1), jnp.float32),
        ],
        scratch_shapes=[
            pltpu.VMEM((_NJ, _BN, _D + 128), jnp.bfloat16),
        ],
        compiler_params=pltpu.CompilerParams(
            dimension_semantics=("arbitrary",)),
    )(z_flat, emb)


def _gather_rows(table, idx_row):
    mesh = plsc.VectorSubcoreMesh(core_axis_name="core",
                                  subcore_axis_name="subcore")

    @pl.kernel(out_type=jax.ShapeDtypeStruct((_N_TOK, _D), table.dtype),
               mesh=mesh)
    def k(x_hbm, i_hbm, o_hbm):
        def body(i_vmem, o_vmem):
            pltpu.sync_copy(x_hbm.at[i_vmem.at[0]], o_vmem)

        pltpu.emit_pipeline(
            body,
            grid=(_N_TOK // _GW,),
            in_specs=[pl.BlockSpec((1, _GW), index_map=lambda i: (0, i))],
            out_specs=[pl.BlockSpec((_GW, _D), index_map=lambda i: (i, 0))],
            core_axis_name=("core", "subcore"),
            dimension_semantics=(pltpu.PARALLEL,),
        )(i_hbm, o_hbm)

    return k(table, idx_row)


def _normalize_body(x_ref, o_ref):
    x = x_ref[...]
    nrm = jnp.sqrt(jnp.sum(x * x, axis=1, keepdims=True))
    o_ref[...] = x / jnp.maximum(nrm, _EPS)


def _normalize_call(x):
    nb = 4
    bm = _N_TOK // nb
    return pl.pallas_call(
        _normalize_body,
        grid=(nb,),
        in_specs=[pl.BlockSpec((bm, _D), lambda i: (i, 0))],
        out_specs=pl.BlockSpec((bm, _D), lambda i: (i, 0)),
        out_shape=jax.ShapeDtypeStruct((_N_TOK, _D), jnp.float32),
    )(x)


def kernel(z, embedding_weight):
    z_flat = z.reshape(_N_TOK, _D)
    idx2, loss = _argmin_call(z_flat, embedding_weight)
    idx = idx2.reshape(_N_TOK)
    z_q = _gather_rows(embedding_weight, idx2.reshape(1, _N_TOK))
    z_qnorm = _normalize_call(z_q)
    return (z_qnorm.reshape(z.shape), loss.reshape(()), idx)


# speedup vs baseline: 1.3432x; 1.3432x over previous
import jax
import jax.numpy as jnp
from jax.experimental import pallas as pl
from jax.experimental.pallas import tpu as pltpu
from jax.experimental.pallas import tpu_sc as plsc

_N_E = 8192
_D = 256
_N_TOK = 4608
_BM = 512
_BN = 1024
_NI = _N_TOK // _BM
_NJ = _N_E // _BN
_GW = 128
_EPS = 1e-12


def _argmin_body(z_ref, et_ref, idx_ref, loss_ref, enb_s, esq_s):
    i = pl.program_id(0)

    @pl.when(i == 0)
    def _prep_codebook():
        for j in range(_NJ):
            et = et_ref[:, j * _BN:(j + 1) * _BN]
            nsq = jnp.sum(et * et, axis=0, keepdims=True)
            m = jnp.maximum(jnp.sqrt(nsq), _EPS)
            ent = et / m
            enb_s[j] = ent.astype(jnp.bfloat16)
            esq_s[j] = jnp.sum(ent * ent, axis=0, keepdims=True)

    zb = z_ref[...]
    nrm = jnp.sqrt(jnp.sum(zb * zb, axis=1, keepdims=True))
    zn = zb / jnp.maximum(nrm, _EPS)
    zsq = jnp.sum(zn * zn, keepdims=True)
    znm2 = (-2.0 * zn).astype(jnp.bfloat16)

    nr = _BM // 128
    rv = [None] * nr
    ri = [None] * nr
    for j in range(_NJ):
        d2 = jax.lax.dot_general(
            znm2, enb_s[j],
            dimension_numbers=(((1,), (0,)), ((), ())),
            preferred_element_type=jnp.float32)
        score = d2 + esq_s[j]
        for r in range(nr):
            rvc, ric = rv[r], ri[r]
            for g in range(_BN // 128):
                sg = score[r * 128:(r + 1) * 128, g * 128:(g + 1) * 128]
                fold = j * (_BN // 128) + g
                if rvc is None:
                    rvc = sg
                    ric = jnp.zeros((128, 128), jnp.int32)
                else:
                    m = sg < rvc
                    rvc = jnp.where(m, sg, rvc)
                    ric = jnp.where(m, jnp.full((128, 128), fold,
                                                jnp.int32), ric)
            rv[r], ri[r] = rvc, ric

    rva = jnp.concatenate(rv, axis=0)
    ria = jnp.concatenate(ri, axis=0)
    rvt = rva.T
    rit = ria.T
    sub = jax.lax.broadcasted_iota(jnp.int32, (128, _BM), 0)
    gidx = rit * 128 + sub
    mv = jnp.min(rvt, axis=0, keepdims=True)
    cand = jnp.where(rvt == mv, gidx, jnp.full((128, _BM), 2**31 - 1,
                                               jnp.int32))
    idx_ref[...] = jnp.min(cand, axis=0, keepdims=True).reshape(1, 1, _BM)
    part = zsq + jnp.sum(mv, keepdims=True)
    prev = jnp.where(i == 0, jnp.zeros((1, 1), jnp.float32),
                     loss_ref[...])
    tot = prev + part
    scale = 1.25 / float(_N_TOK * _D)
    tot = jnp.where(i == _NI - 1, tot * scale, tot)
    loss_ref[...] = tot


def _argmin_call(z_flat, emb):
    return pl.pallas_call(
        _argmin_body,
        grid=(_NI,),
        in_specs=[
            pl.BlockSpec((_BM, _D), lambda i: (i, 0)),
            pl.BlockSpec((_D, _N_E), lambda i: (0, 0)),
        ],
        out_specs=[
            pl.BlockSpec((1, 1, _BM), lambda i: (i, 0, 0)),
            pl.BlockSpec((1, 1), lambda i: (0, 0)),
        ],
        out_shape=[
            jax.ShapeDtypeStruct((_NI, 1, _BM), jnp.int32),
            jax.ShapeDtypeStruct((1, 1), jnp.float32),
        ],
        scratch_shapes=[
            pltpu.VMEM((_NJ, _D, _BN), jnp.bfloat16),
            pltpu.VMEM((_NJ, 1, _BN), jnp.float32),
        ],
        compiler_params=pltpu.CompilerParams(
            dimension_semantics=("arbitrary",)),
    )(z_flat, emb)


def _gather_rows(table, idx_row):
    mesh = plsc.VectorSubcoreMesh(core_axis_name="core",
                                  subcore_axis_name="subcore")

    @pl.kernel(out_type=jax.ShapeDtypeStruct((_N_TOK, _D), table.dtype),
               mesh=mesh)
    def k(x_hbm, i_hbm, o_hbm):
        def body(i_vmem, o_vmem):
            pltpu.sync_copy(x_hbm.at[i_vmem.at[0]], o_vmem)

        pltpu.emit_pipeline(
            body,
            grid=(_N_TOK // _GW,),
            in_specs=[pl.BlockSpec((1, _GW), index_map=lambda i: (0, i))],
            out_specs=[pl.BlockSpec((_GW, _D), index_map=lambda i: (i, 0))],
            core_axis_name=("core", "subcore"),
            dimension_semantics=(pltpu.PARALLEL,),
        )(i_hbm, o_hbm)

    return k(table, idx_row)


def _normalize_body(x_ref, o_ref):
    x = x_ref[...]
    nrm = jnp.sqrt(jnp.sum(x * x, axis=1, keepdims=True))
    o_ref[...] = x / jnp.maximum(nrm, _EPS)


def _normalize_call(x):
    nb = 4
    bm = _N_TOK // nb
    return pl.pallas_call(
        _normalize_body,
        grid=(nb,),
        in_specs=[pl.BlockSpec((bm, _D), lambda i: (i, 0))],
        out_specs=pl.BlockSpec((bm, _D), lambda i: (i, 0)),
        out_shape=jax.ShapeDtypeStruct((_N_TOK, _D), jnp.float32),
    )(x)


def kernel(z, embedding_weight):
    z_flat = z.reshape(_N_TOK, _D)
    idx2, loss = _argmin_call(z_flat, embedding_weight.T)
    idx = idx2.reshape(_N_TOK)
    z_q = _gather_rows(embedding_weight, idx2.reshape(1, _N_TOK))
    z_qnorm = _normalize_call(z_q)
    return (z_qnorm.reshape(z.shape), loss.reshape(()), idx)


# speedup vs baseline: 1.5589x; 1.1606x over previous
import jax
import jax.numpy as jnp
from jax.experimental import pallas as pl
from jax.experimental.pallas import tpu as pltpu
from jax.experimental.pallas import tpu_sc as plsc

_N_E = 8192
_D = 256
_N_TOK = 4608
_BM = 512
_BN = 1024
_NI = _N_TOK // _BM
_NJ = _N_E // _BN
_GW = 128
_EPS = 1e-12


def _argmin_body(z_ref, et_ref, idx_ref, loss_ref, enb_s, esq_s):
    i = pl.program_id(0)

    @pl.when(i == 0)
    def _prep_codebook():
        for j in range(_NJ):
            et = jnp.transpose(et_ref[j * _BN:(j + 1) * _BN, :],
                               (1, 0))
            nsq = jnp.sum(et * et, axis=0, keepdims=True)
            m = jnp.maximum(jnp.sqrt(nsq), _EPS)
            ent = et / m
            enb_s[j] = ent.astype(jnp.bfloat16)
            esq_s[j] = jnp.sum(ent * ent, axis=0, keepdims=True)

    zb = z_ref[...]
    nrm = jnp.sqrt(jnp.sum(zb * zb, axis=1, keepdims=True))
    zn = zb / jnp.maximum(nrm, _EPS)
    zsq = jnp.sum(zn * zn, keepdims=True)
    znm2 = (-2.0 * zn).astype(jnp.bfloat16)

    nr = _BM // 128
    rv = [None] * nr
    ri = [None] * nr
    for j in range(_NJ):
        d2 = jax.lax.dot_general(
            znm2, enb_s[j],
            dimension_numbers=(((1,), (0,)), ((), ())),
            preferred_element_type=jnp.float32)
        score = d2 + esq_s[j]
        for r in range(nr):
            rvc, ric = rv[r], ri[r]
            for g in range(_BN // 128):
                sg = score[r * 128:(r + 1) * 128, g * 128:(g + 1) * 128]
                fold = j * (_BN // 128) + g
                if rvc is None:
                    rvc = sg
                    ric = jnp.zeros((128, 128), jnp.int32)
                else:
                    m = sg < rvc
                    rvc = jnp.where(m, sg, rvc)
                    ric = jnp.where(m, jnp.full((128, 128), fold,
                                                jnp.int32), ric)
            rv[r], ri[r] = rvc, ric

    rva = jnp.concatenate(rv, axis=0)
    ria = jnp.concatenate(ri, axis=0)
    rvt = rva.T
    rit = ria.T
    sub = jax.lax.broadcasted_iota(jnp.int32, (128, _BM), 0)
    gidx = rit * 128 + sub
    mv = jnp.min(rvt, axis=0, keepdims=True)
    cand = jnp.where(rvt == mv, gidx, jnp.full((128, _BM), 2**31 - 1,
                                               jnp.int32))
    idx_ref[...] = jnp.min(cand, axis=0, keepdims=True).reshape(1, 1, _BM)
    part = zsq + jnp.sum(mv, keepdims=True)
    prev = jnp.where(i == 0, jnp.zeros((1, 1), jnp.float32),
                     loss_ref[...])
    tot = prev + part
    scale = 1.25 / float(_N_TOK * _D)
    tot = jnp.where(i == _NI - 1, tot * scale, tot)
    loss_ref[...] = tot


def _argmin_call(z_flat, emb):
    return pl.pallas_call(
        _argmin_body,
        grid=(_NI,),
        in_specs=[
            pl.BlockSpec((_BM, _D), lambda i: (i, 0)),
            pl.BlockSpec((_N_E, _D), lambda i: (0, 0)),
        ],
        out_specs=[
            pl.BlockSpec((1, 1, _BM), lambda i: (i, 0, 0)),
            pl.BlockSpec((1, 1), lambda i: (0, 0)),
        ],
        out_shape=[
            jax.ShapeDtypeStruct((_NI, 1, _BM), jnp.int32),
            jax.ShapeDtypeStruct((1, 1), jnp.float32),
        ],
        scratch_shapes=[
            pltpu.VMEM((_NJ, _D, _BN), jnp.bfloat16),
            pltpu.VMEM((_NJ, 1, _BN), jnp.float32),
        ],
        compiler_params=pltpu.CompilerParams(
            dimension_semantics=("arbitrary",)),
    )(z_flat, emb)


def _gather_rows(table, idx_row):
    mesh = plsc.VectorSubcoreMesh(core_axis_name="core",
                                  subcore_axis_name="subcore")

    @pl.kernel(out_type=jax.ShapeDtypeStruct((_N_TOK, _D), table.dtype),
               mesh=mesh)
    def k(x_hbm, i_hbm, o_hbm):
        def body(i_vmem, o_vmem):
            pltpu.sync_copy(x_hbm.at[i_vmem.at[0]], o_vmem)

        pltpu.emit_pipeline(
            body,
            grid=(_N_TOK // _GW,),
            in_specs=[pl.BlockSpec((1, _GW), index_map=lambda i: (0, i))],
            out_specs=[pl.BlockSpec((_GW, _D), index_map=lambda i: (i, 0))],
            core_axis_name=("core", "subcore"),
            dimension_semantics=(pltpu.PARALLEL,),
        )(i_hbm, o_hbm)

    return k(table, idx_row)


def _normalize_body(x_ref, o_ref):
    x = x_ref[...]
    nrm = jnp.sqrt(jnp.sum(x * x, axis=1, keepdims=True))
    o_ref[...] = x / jnp.maximum(nrm, _EPS)


def _normalize_call(x):
    nb = 4
    bm = _N_TOK // nb
    return pl.pallas_call(
        _normalize_body,
        grid=(nb,),
        in_specs=[pl.BlockSpec((bm, _D), lambda i: (i, 0))],
        out_specs=pl.BlockSpec((bm, _D), lambda i: (i, 0)),
        out_shape=jax.ShapeDtypeStruct((_N_TOK, _D), jnp.float32),
    )(x)


def kernel(z, embedding_weight):
    z_flat = z.reshape(_N_TOK, _D)
    idx2, loss = _argmin_call(z_flat, embedding_weight)
    idx = idx2.reshape(_N_TOK)
    z_q = _gather_rows(embedding_weight, idx2.reshape(1, _N_TOK))
    z_qnorm = _normalize_call(z_q)
    return (z_qnorm.reshape(z.shape), loss.reshape(()), idx)


# speedup vs baseline: 1.5789x; 1.0128x over previous
import jax
import jax.numpy as jnp
from jax.experimental import pallas as pl
from jax.experimental.pallas import tpu as pltpu
from jax.experimental.pallas import tpu_sc as plsc

_N_E = 8192
_D = 256
_N_TOK = 4608
_BM = 512
_BN = 1024
_NI = _N_TOK // _BM
_NJ = _N_E // _BN
_GW = 128
_EPS = 1e-12


def _argmin_body(z_ref, et_ref, idx_ref, loss_ref, en_ref, enb_s, esq_s):
    i = pl.program_id(0)

    @pl.when(i == 0)
    def _prep_codebook():
        for j in range(_NJ):
            et = jnp.transpose(et_ref[j * _BN:(j + 1) * _BN, :],
                               (1, 0))
            nsq = jnp.sum(et * et, axis=0, keepdims=True)
            m = jnp.maximum(jnp.sqrt(nsq), _EPS)
            ent = et / m
            enb_s[j] = ent.astype(jnp.bfloat16)
            esq_s[j] = jnp.sum(ent * ent, axis=0, keepdims=True)
            en_ref[j * _BN:(j + 1) * _BN, :] = jnp.transpose(ent, (1, 0))

    zb = z_ref[...]
    nrm = jnp.sqrt(jnp.sum(zb * zb, axis=1, keepdims=True))
    zn = zb / jnp.maximum(nrm, _EPS)
    zsq = jnp.sum(zn * zn, keepdims=True)
    znm2 = (-2.0 * zn).astype(jnp.bfloat16)

    nr = _BM // 128
    rv = [None] * nr
    ri = [None] * nr
    for j in range(_NJ):
        d2 = jax.lax.dot_general(
            znm2, enb_s[j],
            dimension_numbers=(((1,), (0,)), ((), ())),
            preferred_element_type=jnp.float32)
        score = d2 + esq_s[j]
        for r in range(nr):
            rvc, ric = rv[r], ri[r]
            for g in range(_BN // 128):
                sg = score[r * 128:(r + 1) * 128, g * 128:(g + 1) * 128]
                fold = j * (_BN // 128) + g
                if rvc is None:
                    rvc = sg
                    ric = jnp.zeros((128, 128), jnp.int32)
                else:
                    m = sg < rvc
                    rvc = jnp.where(m, sg, rvc)
                    ric = jnp.where(m, jnp.full((128, 128), fold,
                                                jnp.int32), ric)
            rv[r], ri[r] = rvc, ric

    rva = jnp.concatenate(rv, axis=0)
    ria = jnp.concatenate(ri, axis=0)
    rvt = rva.T
    rit = ria.T
    sub = jax.lax.broadcasted_iota(jnp.int32, (128, _BM), 0)
    gidx = rit * 128 + sub
    mv = jnp.min(rvt, axis=0, keepdims=True)
    cand = jnp.where(rvt == mv, gidx, jnp.full((128, _BM), 2**31 - 1,
                                               jnp.int32))
    idx_ref[...] = jnp.min(cand, axis=0, keepdims=True).reshape(1, 1, _BM)
    part = zsq + jnp.sum(mv, keepdims=True)
    prev = jnp.where(i == 0, jnp.zeros((1, 1), jnp.float32),
                     loss_ref[...])
    tot = prev + part
    scale = 1.25 / float(_N_TOK * _D)
    tot = jnp.where(i == _NI - 1, tot * scale, tot)
    loss_ref[...] = tot


def _argmin_call(z_flat, emb):
    return pl.pallas_call(
        _argmin_body,
        grid=(_NI,),
        in_specs=[
            pl.BlockSpec((_BM, _D), lambda i: (i, 0)),
            pl.BlockSpec((_N_E, _D), lambda i: (0, 0)),
        ],
        out_specs=[
            pl.BlockSpec((1, 1, _BM), lambda i: (i, 0, 0)),
            pl.BlockSpec((1, 1), lambda i: (0, 0)),
            pl.BlockSpec((_N_E, _D), lambda i: (0, 0)),
        ],
        out_shape=[
            jax.ShapeDtypeStruct((_NI, 1, _BM), jnp.int32),
            jax.ShapeDtypeStruct((1, 1), jnp.float32),
            jax.ShapeDtypeStruct((_N_E, _D), jnp.float32),
        ],
        scratch_shapes=[
            pltpu.VMEM((_NJ, _D, _BN), jnp.bfloat16),
            pltpu.VMEM((_NJ, 1, _BN), jnp.float32),
        ],
        compiler_params=pltpu.CompilerParams(
            dimension_semantics=("arbitrary",)),
    )(z_flat, emb)


def _gather_rows(table, idx_row):
    mesh = plsc.VectorSubcoreMesh(core_axis_name="core",
                                  subcore_axis_name="subcore")

    @pl.kernel(out_type=jax.ShapeDtypeStruct((_N_TOK, _D), table.dtype),
               mesh=mesh)
    def k(x_hbm, i_hbm, o_hbm):
        def body(i_vmem, o_vmem):
            pltpu.sync_copy(x_hbm.at[i_vmem.at[0]], o_vmem)

        pltpu.emit_pipeline(
            body,
            grid=(_N_TOK // _GW,),
            in_specs=[pl.BlockSpec((1, _GW), index_map=lambda i: (0, i))],
            out_specs=[pl.BlockSpec((_GW, _D), index_map=lambda i: (i, 0))],
            core_axis_name=("core", "subcore"),
            dimension_semantics=(pltpu.PARALLEL,),
        )(i_hbm, o_hbm)

    return k(table, idx_row)


def kernel(z, embedding_weight):
    z_flat = z.reshape(_N_TOK, _D)
    idx2, loss, en = _argmin_call(z_flat, embedding_weight)
    idx = idx2.reshape(_N_TOK)
    z_qnorm = _gather_rows(en, idx2.reshape(1, _N_TOK))
    return (z_qnorm.reshape(z.shape), loss.reshape(()), idx)
